# baseline (speedup 1.0000x reference)
import functools

import jax
import jax.numpy as jnp
from jax import lax
from jax.experimental import pallas as pl
from jax.experimental.pallas import tpu as pltpu
from jax.experimental.pallas import tpu_sc as plsc

D = 768
B = 4
T = 2048

NC = 2
NS = 16
L = 16
NW = NC * NS
TPW = T // NW
CHT = 8
NCH = TPW // CHT
GR = B * CHT
NBUF = 4
GLEAD = 2


def _embed_body(x_hbm, te_hbm, pe_hbm, out_hbm,
                idx_v, pe0, pe1, grp0, grp1, grp2, grp3,
                isem, psem0, psem1,
                gsem0, gsem1, gsem2, gsem3,
                ssem0, ssem1, ssem2, ssem3):
    cid = lax.axis_index("c")
    sid = lax.axis_index("s")
    wid = sid * NC + cid
    t0 = wid * TPW

    pes = (pe0, pe1)
    psems = (psem0, psem1)
    grps = (grp0, grp1, grp2, grp3)
    gsems = (gsem0, gsem1, gsem2, gsem3)
    ssems = (ssem0, ssem1, ssem2, ssem3)

    idx_cps = []
    for h in range(NCH):
        for b in range(B):
            idx_cps.append(pltpu.async_copy(
                x_hbm.at[pl.ds(b * T + t0 + h * CHT, CHT)],
                idx_v.at[pl.ds(h * GR + b * CHT, CHT)], isem))

    def start_pe(h):
        return pltpu.async_copy(pe_hbm.at[pl.ds(t0 + h * CHT, CHT)],
                                pes[h % 2], psems[h % 2])

    pe_cps = {0: start_pe(0), 1: start_pe(1)}

    for cp in idx_cps:
        cp.wait()

    def start_gather(h):
        return pltpu.async_copy(te_hbm.at[idx_v.at[pl.ds(h * GR, GR)]],
                                grps[h % NBUF], gsems[h % NBUF])

    gathers = {h: start_gather(h) for h in range(GLEAD)}
    stores = {}

    for h in range(NCH):
        grp = grps[h % NBUF]
        gathers[h].wait()
        pe_cps[h].wait()
        pe = pes[h % 2]

        @pl.loop(0, CHT)
        def _(r):
            for c in range(0, D, L):
                v = pe[r, pl.ds(c, L)]
                for b in range(B):
                    plsc.addupdate(grp.at[b * CHT + r, pl.ds(c, L)], v)

        stores[h] = [
            pltpu.async_copy(grp.at[pl.ds(b * CHT, CHT)],
                             out_hbm.at[pl.ds(b * T + t0 + h * CHT, CHT)],
                             ssems[h % NBUF])
            for b in range(B)
        ]
        if h + 2 < NCH:
            pe_cps[h + 2] = start_pe(h + 2)
        nxt = h + GLEAD
        if nxt < NCH:
            for cp in stores.get(nxt - NBUF, []):
                cp.wait()
            gathers[nxt] = start_gather(nxt)

    for h in range(NCH - NBUF, NCH):
        for cp in stores[h]:
            cp.wait()


@jax.jit
def _embed(x_flat, te, pe):
    mesh = plsc.VectorSubcoreMesh(core_axis_name="c", subcore_axis_name="s")
    run = pl.kernel(
        _embed_body,
        out_type=jax.ShapeDtypeStruct((B * T, D), jnp.float32),
        mesh=mesh,
        scratch_types=[
            pltpu.VMEM((B * TPW,), jnp.int32),
            pltpu.VMEM((CHT, D), jnp.float32),
            pltpu.VMEM((CHT, D), jnp.float32),
            pltpu.VMEM((GR, D), jnp.float32),
            pltpu.VMEM((GR, D), jnp.float32),
            pltpu.VMEM((GR, D), jnp.float32),
            pltpu.VMEM((GR, D), jnp.float32),
        ] + [pltpu.SemaphoreType.DMA] * 11,
    )
    return run(x_flat, te, pe)


def kernel(x, te, pe):
    x_flat = x.reshape(B * T).astype(jnp.int32)
    out = _embed(x_flat, te.astype(jnp.float32), pe.astype(jnp.float32))
    return out.reshape(B, T, D)
